# baseline (speedup 1.0000x reference)
import numpy as np
import jax
import jax.numpy as jnp
from jax import lax
from jax.experimental import pallas as pl
from jax.experimental.pallas import tpu as pltpu
from jax.experimental.pallas import tpu_sc as plsc

_THRESH = float(-np.log(0.7))
_N_MIN = 131072
_IGNORE = 255

_B, _C, _H, _W = 8, 19, 512, 512
_CROWS = 128
_NJ = (_H * _W) // (_CROWS * 1024)
_NPIX = _B * _H * _W

_SEL_ROWS = 256
_SEL_COLS = 1024
_NBLK = _NPIX // (_SEL_ROWS * _SEL_COLS)


def _ce_body(logits_ref, labels_ref, out_ref):
    x = logits_ref[0]
    lab = labels_ref[0]
    m = jnp.max(x, axis=0)
    e = jnp.exp(x - m[None])
    s = jnp.sum(e, axis=0)
    lse = jnp.log(s) + m
    cls = lax.broadcasted_iota(jnp.int32, (_C, _CROWS, 1024), 0)
    safe_lab = jnp.where(lab == _IGNORE, 0, lab)
    picked = jnp.sum(jnp.where(cls == safe_lab[None], x, 0.0), axis=0)
    loss = jnp.where(lab == _IGNORE, 0.0, lse - picked)
    out_ref[...] = loss


def _compute_loss(logits, labels):
    lg = logits.reshape(_B, _C, _NJ * _CROWS, 1024)
    lb = labels.reshape(_B, _NJ * _CROWS, 1024)
    return pl.pallas_call(
        _ce_body,
        grid=(_B, _NJ),
        in_specs=[
            pl.BlockSpec((1, _C, _CROWS, 1024), lambda b, j: (b, 0, j, 0)),
            pl.BlockSpec((1, _CROWS, 1024), lambda b, j: (b, j, 0)),
        ],
        out_specs=pl.BlockSpec((_CROWS, 1024), lambda b, j: (b * _NJ + j, 0)),
        out_shape=jax.ShapeDtypeStruct((_B * _NJ * _CROWS, 1024), jnp.float32),
        compiler_params=pltpu.CompilerParams(
            dimension_semantics=("arbitrary", "arbitrary")),
    )(lg, lb)


def _sel_body(loss_ref, out_ref, hist_s, st_s, fs_s, hb_s, cb_s):
    p = pl.program_id(0)
    b = pl.program_id(1)

    @pl.when((p == 0) & (b == 0))
    def _init():
        for j in range(16):
            hist_s[j] = 0
        for j in range(16 * _NBLK):
            hb_s[j] = 0
        for j in range(_NBLK):
            cb_s[j] = 1
        st_s[0] = 0
        st_s[1] = _N_MIN
        st_s[2] = 0
        fs_s[0] = 0.0

    @pl.when((p >= 1) & (b == 0))
    def _select_digit():
        remaining = st_s[1]
        acc = jnp.int32(0)
        dig = jnp.int32(0)
        newrem = remaining
        found = acc > jnp.int32(0)
        for j in range(15, -1, -1):
            hj = hist_s[j]
            hit = jnp.logical_and(jnp.logical_not(found), acc + hj >= remaining)
            dig = jnp.where(hit, jnp.int32(j), dig)
            newrem = jnp.where(hit, remaining - acc, newrem)
            found = jnp.logical_or(found, hit)
            acc = acc + hj
        st_s[0] = jnp.bitwise_or(lax.shift_left(st_s[0], 4), dig)
        st_s[1] = newrem
        for j in range(16):
            hist_s[j] = 0
        for bb in range(_NBLK):
            live = jnp.logical_or(p == 1, cb_s[bb] > 0)
            cb_s[bb] = jnp.where(live, hb_s[bb * 16 + dig], 0)

    @pl.when(p <= 7)
    def _radix_count():
        active = jnp.logical_or(p == 0, cb_s[b] > 0)

        @pl.when(active)
        def _():
            x = loss_ref[...]
            u = lax.bitcast_convert_type(x, jnp.int32)
            shift = (7 - p) * 4
            us = lax.shift_right_logical(u, shift)
            d = jnp.bitwise_and(us, 15)
            hi = lax.shift_right_logical(us, 4)
            in_set = hi == st_s[0]
            base = jnp.where(in_set, d, 16)
            for j in range(16):
                cnt_j = jnp.sum((base == j).astype(jnp.int32))
                hist_s[j] = hist_s[j] + cnt_j
                hb_s[b * 16 + j] = cnt_j

    @pl.when(p == 8)
    def _stats():
        x = loss_ref[...]
        u = lax.bitcast_convert_type(x, jnp.int32)
        gt_t = u > st_s[0]
        fs_s[0] = fs_s[0] + jnp.sum(jnp.where(gt_t, x, 0.0))
        st_s[2] = st_s[2] + jnp.sum(gt_t.astype(jnp.int32))

    @pl.when((p == 8) & (b == _NBLK - 1))
    def _finalize():
        t = jnp.max(lax.bitcast_convert_type(
            jnp.full((1, 128), st_s[0], jnp.int32), jnp.float32))
        l = lax.broadcasted_iota(jnp.int32, (1, 128), 1)
        out_ref[...] = (jnp.where(l == 0, t, 0.0)
                        + jnp.where(l == 1, fs_s[0], 0.0)
                        + jnp.where(l == 2, st_s[2].astype(jnp.float32), 0.0))


def _select(loss2d):
    return pl.pallas_call(
        _sel_body,
        grid=(9, _NBLK),
        in_specs=[pl.BlockSpec((_SEL_ROWS, _SEL_COLS), lambda p, b: (b, 0))],
        out_specs=pl.BlockSpec((1, 128), lambda p, b: (0, 0)),
        out_shape=jax.ShapeDtypeStruct((1, 128), jnp.float32),
        scratch_shapes=[
            pltpu.SMEM((16,), jnp.int32),
            pltpu.SMEM((4,), jnp.int32),
            pltpu.SMEM((2,), jnp.float32),
            pltpu.SMEM((16 * _NBLK,), jnp.int32),
            pltpu.SMEM((_NBLK,), jnp.int32),
        ],
        compiler_params=pltpu.CompilerParams(
            dimension_semantics=("arbitrary", "arbitrary")),
    )(loss2d)


_SC_TILES = 16
_SC_PER_TILE = _NPIX // _SC_TILES
_SC_CHUNK = 32768
_SC_NCHUNK = _SC_PER_TILE // _SC_CHUNK
_SC_NVREG = _SC_CHUNK // 16

def _vsum16(x, lane):
    for k in (1, 2, 4, 8):
        idx = jnp.bitwise_xor(lane, k)
        x = x + x.at[idx].get(mode="promise_in_bounds")
    return x


def _sc_stats_body(loss_hbm, out_hbm, chunk_v, stat_v, res_v, all_v,
                   shared_stats):
    cid = lax.axis_index("c")
    sid = lax.axis_index("s")
    base = sid * _SC_PER_TILE
    lane = lax.iota(jnp.int32, 16)
    thr_vec = jnp.full((16,), jnp.float32(_THRESH))
    onef = jnp.ones((16,), jnp.float32)
    zerof = jnp.zeros((16,), jnp.float32)

    s_thr = zerof
    c_thr = zerof
    for c in range(_SC_NCHUNK):
        pltpu.sync_copy(
            loss_hbm.at[pl.ds(pl.multiple_of(base + c * _SC_CHUNK, 8),
                              _SC_CHUNK)],
            chunk_v)

        def _stat(i, acc):
            a_s, a_c = acc
            v = chunk_v[pl.ds(pl.multiple_of(i * 16, 16), 16)]
            m = v > thr_vec
            return (a_s + jnp.where(m, v, zerof),
                    a_c + jnp.where(m, onef, zerof))
        s_thr, c_thr = lax.fori_loop(0, _SC_NVREG, _stat, (s_thr, c_thr))

    for k, vec in enumerate((s_thr, c_thr)):
        stat_v[...] = vec
        pltpu.sync_copy(
            stat_v,
            shared_stats.at[pl.ds(pl.multiple_of(k * 256 + sid * 16, 8), 16)])
    plsc.subcore_barrier()

    @pl.when((cid == 0) & (sid == 0))
    def _combine():
        pltpu.sync_copy(shared_stats, all_v)
        acc_s = zerof
        acc_c = zerof
        for t in range(16):
            acc_s = acc_s + all_v[pl.ds(t * 16, 16)]
            acc_c = acc_c + all_v[pl.ds(256 + t * 16, 16)]
        tot_s = _vsum16(acc_s, lane)
        tot_c = _vsum16(acc_c, lane)
        res_v[...] = (jnp.where(lane == 0, tot_s, zerof)
                      + jnp.where(lane == 1, tot_c, zerof))
        pltpu.sync_copy(res_v, out_hbm.at[pl.ds(0, 16)])


def _sc_stats(loss_flat):
    f = pl.kernel(
        _sc_stats_body,
        out_type=jax.ShapeDtypeStruct((128,), jnp.float32),
        mesh=plsc.VectorSubcoreMesh(core_axis_name="c", subcore_axis_name="s"),
        scratch_types=[
            pltpu.VMEM((_SC_CHUNK,), jnp.float32),
            pltpu.VMEM((16,), jnp.float32),
            pltpu.VMEM((16,), jnp.float32),
            pltpu.VMEM((512,), jnp.float32),
            pltpu.VMEM_SHARED((512,), jnp.float32),
        ],
    )
    return f(loss_flat)


def _comb_body(sc_ref, tc_ref, out_ref):
    sc = sc_ref[...]
    tc = tc_ref[...]
    l = lax.broadcasted_iota(jnp.int32, (1, 128), 1)
    sum_thr = jnp.sum(jnp.where(l == 0, sc, 0.0))
    cnt_thr = jnp.sum(jnp.where(l == 1, sc, 0.0))
    t = jnp.sum(jnp.where(l == 0, tc, 0.0))
    sum_gt = jnp.sum(jnp.where(l == 1, tc, 0.0))
    cnt_gt = jnp.sum(jnp.where(l == 2, tc, 0.0))
    nmin = jnp.float32(_N_MIN)
    mean_a = sum_thr / jnp.maximum(cnt_thr, 1.0)
    mean_b = (sum_gt + (nmin - cnt_gt) * t) / nmin
    out_ref[...] = jnp.full((1, 128),
                            jnp.where(cnt_thr > nmin, mean_a, mean_b),
                            jnp.float32)


def _combine(sc_stats, tc_part):
    return pl.pallas_call(
        _comb_body,
        out_shape=jax.ShapeDtypeStruct((1, 128), jnp.float32),
    )(sc_stats.reshape(1, 128), tc_part)


def kernel(logits, labels):
    loss = _compute_loss(logits, labels)
    sc_stats = _sc_stats(loss.reshape(-1))
    tc_part = _select(loss)
    out = _combine(sc_stats, tc_part)
    return out[0, 0]


# speedup vs baseline: 1.0048x; 1.0048x over previous
import numpy as np
import jax
import jax.numpy as jnp
from jax import lax
from jax.experimental import pallas as pl
from jax.experimental.pallas import tpu as pltpu
from jax.experimental.pallas import tpu_sc as plsc

_THRESH = float(-np.log(0.7))
_N_MIN = 131072
_IGNORE = 255

_B, _C, _H, _W = 8, 19, 512, 512
_CROWS = 256
_NJ = (_H * _W) // (_CROWS * 1024)
_NPIX = _B * _H * _W

_SEL_ROWS = 256
_SEL_COLS = 1024
_NBLK = _NPIX // (_SEL_ROWS * _SEL_COLS)


def _ce_body(logits_ref, labels_ref, out_ref):
    x = logits_ref[0]
    lab = labels_ref[0]
    m = jnp.max(x, axis=0)
    e = jnp.exp(x - m[None])
    s = jnp.sum(e, axis=0)
    lse = jnp.log(s) + m
    cls = lax.broadcasted_iota(jnp.int32, (_C, _CROWS, 1024), 0)
    safe_lab = jnp.where(lab == _IGNORE, 0, lab)
    picked = jnp.sum(jnp.where(cls == safe_lab[None], x, 0.0), axis=0)
    loss = jnp.where(lab == _IGNORE, 0.0, lse - picked)
    out_ref[...] = loss


def _compute_loss(logits, labels):
    lg = logits.reshape(_B, _C, _NJ * _CROWS, 1024)
    lb = labels.reshape(_B, _NJ * _CROWS, 1024)
    return pl.pallas_call(
        _ce_body,
        grid=(_B, _NJ),
        in_specs=[
            pl.BlockSpec((1, _C, _CROWS, 1024), lambda b, j: (b, 0, j, 0)),
            pl.BlockSpec((1, _CROWS, 1024), lambda b, j: (b, j, 0)),
        ],
        out_specs=pl.BlockSpec((_CROWS, 1024), lambda b, j: (b * _NJ + j, 0)),
        out_shape=jax.ShapeDtypeStruct((_B * _NJ * _CROWS, 1024), jnp.float32),
        compiler_params=pltpu.CompilerParams(
            dimension_semantics=("arbitrary", "arbitrary")),
    )(lg, lb)


def _sel_body(loss_ref, out_ref, hist_s, st_s, fs_s, hb_s, cb_s):
    p = pl.program_id(0)
    b = pl.program_id(1)

    @pl.when((p == 0) & (b == 0))
    def _init():
        for j in range(16):
            hist_s[j] = 0
        for j in range(16 * _NBLK):
            hb_s[j] = 0
        for j in range(_NBLK):
            cb_s[j] = 1
        st_s[0] = 0
        st_s[1] = _N_MIN
        st_s[2] = 0
        fs_s[0] = 0.0

    @pl.when((p >= 1) & (b == 0))
    def _select_digit():
        remaining = st_s[1]
        acc = jnp.int32(0)
        dig = jnp.int32(0)
        newrem = remaining
        found = acc > jnp.int32(0)
        for j in range(15, -1, -1):
            hj = hist_s[j]
            hit = jnp.logical_and(jnp.logical_not(found), acc + hj >= remaining)
            dig = jnp.where(hit, jnp.int32(j), dig)
            newrem = jnp.where(hit, remaining - acc, newrem)
            found = jnp.logical_or(found, hit)
            acc = acc + hj
        st_s[0] = jnp.bitwise_or(lax.shift_left(st_s[0], 4), dig)
        st_s[1] = newrem
        for j in range(16):
            hist_s[j] = 0
        for bb in range(_NBLK):
            live = jnp.logical_or(p == 1, cb_s[bb] > 0)
            cb_s[bb] = jnp.where(live, hb_s[bb * 16 + dig], 0)

    @pl.when(p <= 7)
    def _radix_count():
        active = jnp.logical_or(p == 0, cb_s[b] > 0)

        @pl.when(active)
        def _():
            x = loss_ref[...]
            u = lax.bitcast_convert_type(x, jnp.int32)
            shift = (7 - p) * 4
            us = lax.shift_right_logical(u, shift)
            d = jnp.bitwise_and(us, 15)
            hi = lax.shift_right_logical(us, 4)
            in_set = hi == st_s[0]
            base = jnp.where(in_set, d, 16)
            for j in range(16):
                cnt_j = jnp.sum((base == j).astype(jnp.int32))
                hist_s[j] = hist_s[j] + cnt_j
                hb_s[b * 16 + j] = cnt_j

    @pl.when(p == 8)
    def _stats():
        x = loss_ref[...]
        u = lax.bitcast_convert_type(x, jnp.int32)
        gt_t = u > st_s[0]
        fs_s[0] = fs_s[0] + jnp.sum(jnp.where(gt_t, x, 0.0))
        st_s[2] = st_s[2] + jnp.sum(gt_t.astype(jnp.int32))

    @pl.when((p == 8) & (b == _NBLK - 1))
    def _finalize():
        t = jnp.max(lax.bitcast_convert_type(
            jnp.full((1, 128), st_s[0], jnp.int32), jnp.float32))
        l = lax.broadcasted_iota(jnp.int32, (1, 128), 1)
        out_ref[...] = (jnp.where(l == 0, t, 0.0)
                        + jnp.where(l == 1, fs_s[0], 0.0)
                        + jnp.where(l == 2, st_s[2].astype(jnp.float32), 0.0))


def _select(loss2d):
    return pl.pallas_call(
        _sel_body,
        grid=(9, _NBLK),
        in_specs=[pl.BlockSpec((_SEL_ROWS, _SEL_COLS), lambda p, b: (b, 0))],
        out_specs=pl.BlockSpec((1, 128), lambda p, b: (0, 0)),
        out_shape=jax.ShapeDtypeStruct((1, 128), jnp.float32),
        scratch_shapes=[
            pltpu.SMEM((16,), jnp.int32),
            pltpu.SMEM((4,), jnp.int32),
            pltpu.SMEM((2,), jnp.float32),
            pltpu.SMEM((16 * _NBLK,), jnp.int32),
            pltpu.SMEM((_NBLK,), jnp.int32),
        ],
        compiler_params=pltpu.CompilerParams(
            dimension_semantics=("arbitrary", "arbitrary")),
    )(loss2d)


_SC_TILES = 16
_SC_PER_TILE = _NPIX // _SC_TILES
_SC_CHUNK = 32768
_SC_NCHUNK = _SC_PER_TILE // _SC_CHUNK
_SC_NVREG = _SC_CHUNK // 16

def _vsum16(x, lane):
    for k in (1, 2, 4, 8):
        idx = jnp.bitwise_xor(lane, k)
        x = x + x.at[idx].get(mode="promise_in_bounds")
    return x


def _sc_stats_body(loss_hbm, out_hbm, chunk_v, stat_v, res_v, all_v,
                   shared_stats):
    cid = lax.axis_index("c")
    sid = lax.axis_index("s")
    base = sid * _SC_PER_TILE
    lane = lax.iota(jnp.int32, 16)
    thr_vec = jnp.full((16,), jnp.float32(_THRESH))
    onef = jnp.ones((16,), jnp.float32)
    zerof = jnp.zeros((16,), jnp.float32)

    s_thr = zerof
    c_thr = zerof
    for c in range(_SC_NCHUNK):
        pltpu.sync_copy(
            loss_hbm.at[pl.ds(pl.multiple_of(base + c * _SC_CHUNK, 8),
                              _SC_CHUNK)],
            chunk_v)

        def _stat(i, acc):
            a_s, a_c = acc
            v = chunk_v[pl.ds(pl.multiple_of(i * 16, 16), 16)]
            m = v > thr_vec
            return (a_s + jnp.where(m, v, zerof),
                    a_c + jnp.where(m, onef, zerof))
        s_thr, c_thr = lax.fori_loop(0, _SC_NVREG, _stat, (s_thr, c_thr))

    for k, vec in enumerate((s_thr, c_thr)):
        stat_v[...] = vec
        pltpu.sync_copy(
            stat_v,
            shared_stats.at[pl.ds(pl.multiple_of(k * 256 + sid * 16, 8), 16)])
    plsc.subcore_barrier()

    @pl.when((cid == 0) & (sid == 0))
    def _combine():
        pltpu.sync_copy(shared_stats, all_v)
        acc_s = zerof
        acc_c = zerof
        for t in range(16):
            acc_s = acc_s + all_v[pl.ds(t * 16, 16)]
            acc_c = acc_c + all_v[pl.ds(256 + t * 16, 16)]
        tot_s = _vsum16(acc_s, lane)
        tot_c = _vsum16(acc_c, lane)
        res_v[...] = (jnp.where(lane == 0, tot_s, zerof)
                      + jnp.where(lane == 1, tot_c, zerof))
        pltpu.sync_copy(res_v, out_hbm.at[pl.ds(0, 16)])


def _sc_stats(loss_flat):
    f = pl.kernel(
        _sc_stats_body,
        out_type=jax.ShapeDtypeStruct((128,), jnp.float32),
        mesh=plsc.VectorSubcoreMesh(core_axis_name="c", subcore_axis_name="s"),
        scratch_types=[
            pltpu.VMEM((_SC_CHUNK,), jnp.float32),
            pltpu.VMEM((16,), jnp.float32),
            pltpu.VMEM((16,), jnp.float32),
            pltpu.VMEM((512,), jnp.float32),
            pltpu.VMEM_SHARED((512,), jnp.float32),
        ],
    )
    return f(loss_flat)


def _comb_body(sc_ref, tc_ref, out_ref):
    sc = sc_ref[...]
    tc = tc_ref[...]
    l = lax.broadcasted_iota(jnp.int32, (1, 128), 1)
    sum_thr = jnp.sum(jnp.where(l == 0, sc, 0.0))
    cnt_thr = jnp.sum(jnp.where(l == 1, sc, 0.0))
    t = jnp.sum(jnp.where(l == 0, tc, 0.0))
    sum_gt = jnp.sum(jnp.where(l == 1, tc, 0.0))
    cnt_gt = jnp.sum(jnp.where(l == 2, tc, 0.0))
    nmin = jnp.float32(_N_MIN)
    mean_a = sum_thr / jnp.maximum(cnt_thr, 1.0)
    mean_b = (sum_gt + (nmin - cnt_gt) * t) / nmin
    out_ref[...] = jnp.full((1, 128),
                            jnp.where(cnt_thr > nmin, mean_a, mean_b),
                            jnp.float32)


def _combine(sc_stats, tc_part):
    return pl.pallas_call(
        _comb_body,
        out_shape=jax.ShapeDtypeStruct((1, 128), jnp.float32),
    )(sc_stats.reshape(1, 128), tc_part)


def kernel(logits, labels):
    loss = _compute_loss(logits, labels)
    sc_stats = _sc_stats(loss.reshape(-1))
    tc_part = _select(loss)
    out = _combine(sc_stats, tc_part)
    return out[0, 0]
